# baseline (speedup 1.0000x reference)
import functools

import jax
import jax.numpy as jnp
from jax import lax
from jax.experimental import pallas as pl
from jax.experimental.pallas import tpu as pltpu
from jax.experimental.pallas import tpu_sc as plsc

_NUM_CORES = 2
_NUM_SUBCORES = 16
_NW = _NUM_CORES * _NUM_SUBCORES

_V = 1000000
_D = 64
_B = 819200
_T = 200
_BATCH = 4096

_CHUNK = 128
_NBUF = 8
_DEPTH = 6

_NBLK = _V // 128
_TAIL = _V - _NBLK * 128


def _mesh():
    return plsc.VectorSubcoreMesh(core_axis_name="c", subcore_axis_name="s")


def _wid():
    return lax.axis_index("s") * _NUM_CORES + lax.axis_index("c")


def _compact_table(tablet, tail128):
    @functools.partial(
        pl.kernel,
        mesh=_mesh(),
        out_type=jax.ShapeDtypeStruct((_V * _D,), jnp.float32),
        scratch_types=[
            pltpu.VMEM((_D, 128), jnp.float32),
            pltpu.VMEM((128 * _D,), jnp.float32),
        ],
        compiler_params=pltpu.CompilerParams(needs_layout_passes=False),
    )
    def k(tab_hbm, tail_hbm, out_hbm, in_v, out_v):
        iota16 = jnp.arange(16, dtype=jnp.int32)
        wid = _wid()
        per = _NBLK // _NW
        extra = _NBLK - per * _NW
        base = wid * per + jnp.minimum(wid, extra)
        cnt = per + jnp.where(wid < extra, 1, 0)

        def transpose_block():
            for i in range(128):
                cols = jnp.full((16,), i, dtype=jnp.int32)
                for K in range(4):
                    rows = iota16 + (K * 16)
                    v = plsc.load_gather(in_v, [rows, cols])
                    out_v[pl.ds(i * _D + K * 16, 16)] = v

        def body(j, carry):
            c = base + j
            pltpu.sync_copy(tab_hbm.at[:, pl.ds(c * 128, 128)], in_v)
            transpose_block()
            pltpu.sync_copy(out_v, out_hbm.at[pl.ds(c * (128 * _D), 128 * _D)])
            return carry

        lax.fori_loop(0, cnt, body, 0)

        @pl.when(wid == _NW - 1)
        def _():
            pltpu.sync_copy(tail_hbm, in_v)
            transpose_block()
            pltpu.sync_copy(
                out_v.at[pl.ds(_TAIL * _D, _TAIL * _D)],
                out_hbm.at[pl.ds(_NBLK * 128 * _D, _TAIL * _D)],
            )

    return k(tablet, tail128)


def _gather(indices_2d, table_2d):
    n_rows, C = indices_2d.shape
    n_chunks = n_rows // _NW
    b_per_w = n_chunks * C

    @functools.partial(
        pl.kernel,
        mesh=_mesh(),
        out_type=jax.ShapeDtypeStruct((_B, _D), jnp.float32),
        scratch_types=[
            pltpu.VMEM((n_chunks, C), jnp.int32),
            pltpu.VMEM((_NBUF, C, _D), jnp.float32),
        ] + [pltpu.SemaphoreType.DMA] * _NBUF,
        compiler_params=pltpu.CompilerParams(use_tc_tiling_on_sc=False),
    )
    def k(idx_hbm, table_hbm, out_hbm, idx_v, rows_v, *sems):
        wid = _wid()
        base = wid * b_per_w

        pltpu.sync_copy(idx_hbm.at[pl.ds(wid * n_chunks, n_chunks)], idx_v)

        def start_gather(i, b):
            pltpu.async_copy(table_hbm.at[idx_v.at[i]], rows_v.at[b], sems[b])

        def wait_gather(i, b):
            pltpu.make_async_copy(
                table_hbm.at[idx_v.at[i]], rows_v.at[b], sems[b]
            ).wait()

        def start_store(i, b):
            pltpu.async_copy(
                rows_v.at[b], out_hbm.at[pl.ds(base + i * C, C)], sems[b]
            )

        def wait_store(i, b):
            pltpu.make_async_copy(
                rows_v.at[b], out_hbm.at[pl.ds(base + i * C, C)], sems[b]
            ).wait()

        for i in range(_DEPTH):
            start_gather(i, i)

        for i in range(_NBUF - _DEPTH):
            wait_gather(i, i)
            start_store(i, i)
            start_gather(i + _DEPTH, (i + _DEPTH) % _NBUF)

        lo = _NBUF - _DEPTH

        def body(j, carry):
            for u in range(_NBUF):
                i = lo + j * _NBUF + u
                b = (lo + u) % _NBUF
                bg = (lo + u + _DEPTH) % _NBUF
                wait_gather(i, b)
                start_store(i, b)
                wait_store(i + _DEPTH - _NBUF, bg)
                start_gather(i + _DEPTH, bg)
            return carry

        lax.fori_loop(0, (n_chunks - lo - _DEPTH) // _NBUF, body, 0)

        for u in range(_DEPTH):
            i = n_chunks - _DEPTH + u
            wait_gather(i, i % _NBUF)
            start_store(i, i % _NBUF)

        for u in range(_NBUF):
            i = n_chunks - _NBUF + u
            wait_store(i, i % _NBUF)

    return k(indices_2d, table_2d)


def _retile_output(flat):
    n_patches = _T * (_BATCH // 128) // _NW

    @functools.partial(
        pl.kernel,
        mesh=_mesh(),
        out_type=jax.ShapeDtypeStruct((_T, _D, _BATCH), jnp.float32),
        scratch_types=[
            pltpu.VMEM((128 * _D,), jnp.float32),
            pltpu.VMEM((_D, 128), jnp.float32),
        ],
        compiler_params=pltpu.CompilerParams(needs_layout_passes=False),
    )
    def k(in_hbm, out_hbm, in_v, out_v):
        iota64 = jnp.arange(16, dtype=jnp.int32) * _D
        wid = _wid()
        base = wid * n_patches

        def body(j, carry):
            pid = base + j
            t = pid // 32
            b0 = (pid % 32) * 128
            pltpu.sync_copy(in_hbm.at[pl.ds(pid * (128 * _D), 128 * _D)], in_v)
            for kk in range(_D):
                for K in range(8):
                    idx = iota64 + (K * 16 * _D + kk)
                    v = plsc.load_gather(in_v, [idx])
                    out_v[kk, pl.ds(K * 16, 16)] = v
            pltpu.sync_copy(out_v, out_hbm.at[t, :, pl.ds(b0, 128)])
            return carry

        lax.fori_loop(0, n_patches, body, 0)

    return k(flat)


def kernel(indices, table):
    tablet = table.T
    tail128 = table[_V - 128:, :].T
    tlin = _compact_table(tablet, tail128)
    idx_t = indices.T.reshape(-1, _CHUNK).astype(jnp.int32)
    out_t = _gather(idx_t, tlin.reshape(_V, _D))
    out3 = _retile_output(out_t.reshape(-1))
    return out3.transpose(2, 0, 1)


# speedup vs baseline: 1.5014x; 1.5014x over previous
import functools

import jax
import jax.numpy as jnp
from jax import lax
from jax.experimental import pallas as pl
from jax.experimental.pallas import tpu as pltpu
from jax.experimental.pallas import tpu_sc as plsc

_NUM_CORES = 2
_NUM_SUBCORES = 16
_NW = _NUM_CORES * _NUM_SUBCORES

_V = 1000000
_D = 64
_B = 819200
_T = 200
_BATCH = 4096

_CHUNK = 128
_NBUF = 8
_DEPTH = 6

_NBLK = _V // 128
_TAIL = _V - _NBLK * 128


def _mesh():
    return plsc.VectorSubcoreMesh(core_axis_name="c", subcore_axis_name="s")


def _wid():
    return lax.axis_index("s") * _NUM_CORES + lax.axis_index("c")


def _compact_table(tablet, tail128):
    @functools.partial(
        pl.kernel,
        mesh=_mesh(),
        out_type=jax.ShapeDtypeStruct((_V * _D,), jnp.float32),
        scratch_types=[
            pltpu.VMEM((_D, 128), jnp.float32),
            pltpu.VMEM((_D, 128), jnp.float32),
            pltpu.VMEM((128 * _D,), jnp.float32),
            pltpu.VMEM((128 * _D,), jnp.float32),
        ] + [pltpu.SemaphoreType.DMA] * 4,
        compiler_params=pltpu.CompilerParams(needs_layout_passes=False),
    )
    def k(tab_hbm, tail_hbm, out_hbm, in_v0, in_v1, out_v0, out_v1,
          si0, si1, so0, so1):
        iota64 = jnp.arange(16, dtype=jnp.int32) * _D
        wid = _wid()
        per = _NBLK // _NW
        base = wid * per

        inv = (in_v0, in_v1)
        outv = (out_v0, out_v1)
        si = (si0, si1)
        so = (so0, so1)

        def start_in(c, b):
            pltpu.async_copy(tab_hbm.at[:, pl.ds(c * 128, 128)], inv[b], si[b])

        def wait_in(c, b):
            pltpu.make_async_copy(
                tab_hbm.at[:, pl.ds(c * 128, 128)], inv[b], si[b]
            ).wait()

        def start_out(c, b):
            pltpu.async_copy(
                outv[b], out_hbm.at[pl.ds(c * (128 * _D), 128 * _D)], so[b]
            )

        def wait_out(c, b):
            pltpu.make_async_copy(
                outv[b], out_hbm.at[pl.ds(c * (128 * _D), 128 * _D)], so[b]
            ).wait()

        def transpose_block(b):
            inr = inv[b]
            outr = outv[b]
            for k in range(_D):
                for I in range(8):
                    v = inr[k, pl.ds(16 * I, 16)]
                    plsc.store_scatter(outr, [iota64 + (I * 16 * _D + k)], v)

        start_in(base + 0, 0)
        start_in(base + 1, 1)
        for j in range(2):
            wait_in(base + j, j)
            transpose_block(j)
            start_out(base + j, j)
            start_in(base + j + 2, j)

        def body(m, carry):
            for u in range(2):
                j = base + 2 + m * 2 + u
                wait_in(j, u)
                wait_out(j - 2, u)
                transpose_block(u)
                start_out(j, u)

                @pl.when(j + 2 < base + per)
                def _():
                    start_in(j + 2, u)

            return carry

        lax.fori_loop(0, (per - 2) // 2, body, 0)
        wait_out(base + per - 2, 0)
        wait_out(base + per - 1, 1)

        @pl.when(wid < _NBLK - per * _NW)
        def _():
            c = _NW * per + wid
            pltpu.sync_copy(tab_hbm.at[:, pl.ds(c * 128, 128)], in_v0)
            transpose_block(0)
            pltpu.sync_copy(
                out_v0, out_hbm.at[pl.ds(c * (128 * _D), 128 * _D)]
            )

        @pl.when(wid == _NW - 1)
        def _():
            pltpu.sync_copy(tail_hbm, in_v0)
            transpose_block(0)
            pltpu.sync_copy(
                out_v0.at[pl.ds(_TAIL * _D, _TAIL * _D)],
                out_hbm.at[pl.ds(_NBLK * 128 * _D, _TAIL * _D)],
            )

    return k(tablet, tail128)


def _gather(indices_2d, table_2d):
    n_rows, C = indices_2d.shape
    n_chunks = n_rows // _NW
    b_per_w = n_chunks * C

    @functools.partial(
        pl.kernel,
        mesh=_mesh(),
        out_type=jax.ShapeDtypeStruct((_B, _D), jnp.float32),
        scratch_types=[
            pltpu.VMEM((n_chunks, C), jnp.int32),
            pltpu.VMEM((_NBUF, C, _D), jnp.float32),
        ] + [pltpu.SemaphoreType.DMA] * _NBUF,
        compiler_params=pltpu.CompilerParams(use_tc_tiling_on_sc=False),
    )
    def k(idx_hbm, table_hbm, out_hbm, idx_v, rows_v, *sems):
        wid = _wid()
        base = wid * b_per_w

        pltpu.sync_copy(idx_hbm.at[pl.ds(wid * n_chunks, n_chunks)], idx_v)

        def start_gather(i, b):
            pltpu.async_copy(table_hbm.at[idx_v.at[i]], rows_v.at[b], sems[b])

        def wait_gather(i, b):
            pltpu.make_async_copy(
                table_hbm.at[idx_v.at[i]], rows_v.at[b], sems[b]
            ).wait()

        def start_store(i, b):
            pltpu.async_copy(
                rows_v.at[b], out_hbm.at[pl.ds(base + i * C, C)], sems[b]
            )

        def wait_store(i, b):
            pltpu.make_async_copy(
                rows_v.at[b], out_hbm.at[pl.ds(base + i * C, C)], sems[b]
            ).wait()

        for i in range(_DEPTH):
            start_gather(i, i)

        for i in range(_NBUF - _DEPTH):
            wait_gather(i, i)
            start_store(i, i)
            start_gather(i + _DEPTH, (i + _DEPTH) % _NBUF)

        lo = _NBUF - _DEPTH

        def body(j, carry):
            for u in range(_NBUF):
                i = lo + j * _NBUF + u
                b = (lo + u) % _NBUF
                bg = (lo + u + _DEPTH) % _NBUF
                wait_gather(i, b)
                start_store(i, b)
                wait_store(i + _DEPTH - _NBUF, bg)
                start_gather(i + _DEPTH, bg)
            return carry

        lax.fori_loop(0, (n_chunks - lo - _DEPTH) // _NBUF, body, 0)

        for u in range(_DEPTH):
            i = n_chunks - _DEPTH + u
            wait_gather(i, i % _NBUF)
            start_store(i, i % _NBUF)

        for u in range(_NBUF):
            i = n_chunks - _NBUF + u
            wait_store(i, i % _NBUF)

    return k(indices_2d, table_2d)


def _retile_output(flat):
    n_patches = _T * (_BATCH // 128) // _NW

    @functools.partial(
        pl.kernel,
        mesh=_mesh(),
        out_type=jax.ShapeDtypeStruct((_T, _D, _BATCH), jnp.float32),
        scratch_types=[
            pltpu.VMEM((128 * _D,), jnp.float32),
            pltpu.VMEM((128 * _D,), jnp.float32),
            pltpu.VMEM((1, _D, 128), jnp.float32),
            pltpu.VMEM((1, _D, 128), jnp.float32),
        ] + [pltpu.SemaphoreType.DMA] * 4,
        compiler_params=pltpu.CompilerParams(needs_layout_passes=False),
    )
    def k(in_hbm, out_hbm, in_v0, in_v1, out_v0, out_v1, si0, si1, so0, so1):
        iota16 = jnp.arange(16, dtype=jnp.int32)
        zero16 = jnp.zeros((16,), dtype=jnp.int32)
        wid = _wid()
        base = wid * n_patches

        inv = (in_v0, in_v1)
        outv = (out_v0, out_v1)
        si = (si0, si1)
        so = (so0, so1)

        def tb(pid):
            return pid // 32, (pid % 32) * 128

        def start_in(pid, b):
            pltpu.async_copy(
                in_hbm.at[pl.ds(pid * (128 * _D), 128 * _D)], inv[b], si[b]
            )

        def wait_in(pid, b):
            pltpu.make_async_copy(
                in_hbm.at[pl.ds(pid * (128 * _D), 128 * _D)], inv[b], si[b]
            ).wait()

        def start_out(pid, b):
            t, b0 = tb(pid)
            pltpu.async_copy(
                outv[b], out_hbm.at[pl.ds(t, 1), :, pl.ds(b0, 128)], so[b]
            )

        def wait_out(pid, b):
            t, b0 = tb(pid)
            pltpu.make_async_copy(
                outv[b], out_hbm.at[pl.ds(t, 1), :, pl.ds(b0, 128)], so[b]
            ).wait()

        def transpose_patch(b):
            inr = inv[b]
            outr = outv[b]
            for bb in range(128):
                cols = jnp.full((16,), bb, dtype=jnp.int32)
                for J in range(4):
                    v = inr[pl.ds(bb * _D + 16 * J, 16)]
                    plsc.store_scatter(outr, [zero16, iota16 + 16 * J, cols], v)

        start_in(base + 0, 0)
        start_in(base + 1, 1)
        for j in range(2):
            wait_in(base + j, j)
            transpose_patch(j)
            start_out(base + j, j)
            start_in(base + j + 2, j)

        def body(m, carry):
            for u in range(2):
                pid = base + 2 + m * 2 + u
                wait_in(pid, u)
                wait_out(pid - 2, u)
                transpose_patch(u)
                start_out(pid, u)

                @pl.when(pid + 2 < base + n_patches)
                def _():
                    start_in(pid + 2, u)

            return carry

        lax.fori_loop(0, (n_patches - 2) // 2, body, 0)
        wait_out(base + n_patches - 2, 0)
        wait_out(base + n_patches - 1, 1)

    return k(flat)


def kernel(indices, table):
    tablet = table.T
    tail128 = table[_V - 128:, :].T
    tlin = _compact_table(tablet, tail128)
    idx_t = indices.T.reshape(-1, _CHUNK).astype(jnp.int32)
    out_t = _gather(idx_t, tlin.reshape(_V, _D))
    out3 = _retile_output(out_t.reshape(-1))
    return out3.transpose(2, 0, 1)


# speedup vs baseline: 2.9009x; 1.9322x over previous
import functools

import jax
import jax.numpy as jnp
from jax import lax
from jax.experimental import pallas as pl
from jax.experimental.pallas import tpu as pltpu
from jax.experimental.pallas import tpu_sc as plsc

_NUM_CORES = 2
_NUM_SUBCORES = 16
_NW = _NUM_CORES * _NUM_SUBCORES
_CHUNK = 128
_NBUF = 8
_DEPTH = 6


def _gather_flat(indices_2d, table):
    n_rows, C = indices_2d.shape
    D = table.shape[1]
    B = n_rows * C
    assert n_rows % _NW == 0
    n_chunks = n_rows // _NW
    b_per_w = n_chunks * C
    assert (n_chunks - (_NBUF - _DEPTH) - _DEPTH) % _NBUF == 0

    mesh = plsc.VectorSubcoreMesh(core_axis_name="c", subcore_axis_name="s")

    @functools.partial(
        pl.kernel,
        mesh=mesh,
        out_type=jax.ShapeDtypeStruct((B, D), jnp.float32),
        scratch_types=[
            pltpu.VMEM((n_chunks, C), jnp.int32),
            pltpu.VMEM((_NBUF, C, D), jnp.float32),
        ] + [pltpu.SemaphoreType.DMA] * _NBUF,
        compiler_params=pltpu.CompilerParams(use_tc_tiling_on_sc=False),
    )
    def k(idx_hbm, table_hbm, out_hbm, idx_v, rows_v, *sems):
        wid = lax.axis_index("s") * _NUM_CORES + lax.axis_index("c")
        base = wid * b_per_w

        pltpu.sync_copy(idx_hbm.at[pl.ds(wid * n_chunks, n_chunks)], idx_v)

        def start_gather(i, b):
            pltpu.async_copy(table_hbm.at[idx_v.at[i]], rows_v.at[b], sems[b])

        def wait_gather(i, b):
            pltpu.make_async_copy(
                table_hbm.at[idx_v.at[i]], rows_v.at[b], sems[b]
            ).wait()

        def start_store(i, b):
            pltpu.async_copy(
                rows_v.at[b], out_hbm.at[pl.ds(base + i * C, C)], sems[b]
            )

        def wait_store(i, b):
            pltpu.make_async_copy(
                rows_v.at[b], out_hbm.at[pl.ds(base + i * C, C)], sems[b]
            ).wait()

        for i in range(_DEPTH):
            start_gather(i, i)

        for i in range(_NBUF - _DEPTH):
            wait_gather(i, i)
            start_store(i, i)
            start_gather(i + _DEPTH, (i + _DEPTH) % _NBUF)

        lo = _NBUF - _DEPTH

        def body(j, carry):
            for u in range(_NBUF):
                i = lo + j * _NBUF + u
                b = (lo + u) % _NBUF
                bg = (lo + u + _DEPTH) % _NBUF
                wait_gather(i, b)
                start_store(i, b)
                wait_store(i + _DEPTH - _NBUF, bg)
                start_gather(i + _DEPTH, bg)
            return carry

        n_steady = (n_chunks - lo - _DEPTH) // _NBUF
        lax.fori_loop(0, n_steady, body, 0)

        for u in range(_DEPTH):
            i = n_chunks - _DEPTH + u
            wait_gather(i, i % _NBUF)
            start_store(i, i % _NBUF)

        for u in range(_NBUF):
            i = n_chunks - _NBUF + u
            wait_store(i, i % _NBUF)

    return k(indices_2d, table)


def kernel(indices, table):
    shape = indices.shape
    flat = indices.reshape(-1, _CHUNK).astype(jnp.int32)
    out = _gather_flat(flat, table)
    return out.reshape(*shape, table.shape[1])


# speedup vs baseline: 2.9103x; 1.0032x over previous
import functools

import jax
import jax.numpy as jnp
from jax import lax
from jax.experimental import pallas as pl
from jax.experimental.pallas import tpu as pltpu
from jax.experimental.pallas import tpu_sc as plsc

_NUM_CORES = 2
_NUM_SUBCORES = 16
_NW = _NUM_CORES * _NUM_SUBCORES
_CHUNK = 200
_NBUF = 4
_DEPTH = 3


def _gather_flat(indices_2d, table):
    n_rows, C = indices_2d.shape
    D = table.shape[1]
    B = n_rows * C
    assert n_rows % _NW == 0
    n_chunks = n_rows // _NW
    b_per_w = n_chunks * C
    assert (n_chunks - (_NBUF - _DEPTH) - _DEPTH) % _NBUF == 0

    mesh = plsc.VectorSubcoreMesh(core_axis_name="c", subcore_axis_name="s")

    @functools.partial(
        pl.kernel,
        mesh=mesh,
        out_type=jax.ShapeDtypeStruct((B, D), jnp.float32),
        scratch_types=[
            pltpu.VMEM((n_chunks, C), jnp.int32),
            pltpu.VMEM((_NBUF, C, D), jnp.float32),
        ] + [pltpu.SemaphoreType.DMA] * _NBUF,
        compiler_params=pltpu.CompilerParams(use_tc_tiling_on_sc=False),
    )
    def k(idx_hbm, table_hbm, out_hbm, idx_v, rows_v, *sems):
        wid = lax.axis_index("s") * _NUM_CORES + lax.axis_index("c")
        base = wid * b_per_w

        pltpu.sync_copy(idx_hbm.at[pl.ds(wid * n_chunks, n_chunks)], idx_v)

        def start_gather(i, b):
            pltpu.async_copy(table_hbm.at[idx_v.at[i]], rows_v.at[b], sems[b])

        def wait_gather(i, b):
            pltpu.make_async_copy(
                table_hbm.at[idx_v.at[i]], rows_v.at[b], sems[b]
            ).wait()

        def start_store(i, b):
            pltpu.async_copy(
                rows_v.at[b], out_hbm.at[pl.ds(base + i * C, C)], sems[b]
            )

        def wait_store(i, b):
            pltpu.make_async_copy(
                rows_v.at[b], out_hbm.at[pl.ds(base + i * C, C)], sems[b]
            ).wait()

        for i in range(_DEPTH):
            start_gather(i, i)

        for i in range(_NBUF - _DEPTH):
            wait_gather(i, i)
            start_store(i, i)
            start_gather(i + _DEPTH, (i + _DEPTH) % _NBUF)

        lo = _NBUF - _DEPTH

        def body(j, carry):
            for u in range(_NBUF):
                i = lo + j * _NBUF + u
                b = (lo + u) % _NBUF
                bg = (lo + u + _DEPTH) % _NBUF
                wait_gather(i, b)
                start_store(i, b)
                wait_store(i + _DEPTH - _NBUF, bg)
                start_gather(i + _DEPTH, bg)
            return carry

        n_steady = (n_chunks - lo - _DEPTH) // _NBUF
        lax.fori_loop(0, n_steady, body, 0)

        for u in range(_DEPTH):
            i = n_chunks - _DEPTH + u
            wait_gather(i, i % _NBUF)
            start_store(i, i % _NBUF)

        for u in range(_NBUF):
            i = n_chunks - _NBUF + u
            wait_store(i, i % _NBUF)

    return k(indices_2d, table)


def kernel(indices, table):
    return _gather_flat(indices.astype(jnp.int32), table)
